# Initial kernel scaffold; baseline (speedup 1.0000x reference)
import jax
import jax.numpy as jnp
from jax.experimental import pallas as pl


def kernel(user_table, item_table, edge_index, edge_weight):
    raise NotImplementedError("write your pallas kernel here")


# speedup vs baseline: 3.5121x; 3.5121x over previous
import jax
import jax.numpy as jnp
from jax import lax
from jax.experimental import pallas as pl
from jax.experimental.pallas import tpu as pltpu
from jax.experimental.pallas import tpu_sc as plsc

N_USERS = 50000
N_ITEMS = 50000
N_NODES = N_USERS + N_ITEMS
D = 32
HALF = N_NODES // 2
E = 1600000
GROUP = 128
NGROUPS = E // GROUP
NS = 16
GB, GR = divmod(NGROUPS, NS)

CHUNK = 400
NCHUNKS = HALF // CHUNK
CB, CR = divmod(NCHUNKS, NS)


def _make_layer(last: bool):
    mesh = plsc.VectorSubcoreMesh(core_axis_name="c", subcore_axis_name="s")
    if last:
        out_type = jax.ShapeDtypeStruct((N_NODES, D), jnp.float32)
    else:
        out_type = (jax.ShapeDtypeStruct((N_NODES, D), jnp.float32),
                    jax.ShapeDtypeStruct((N_NODES, D), jnp.float32))
    scratch = [
        pltpu.VMEM_SHARED((HALF + 8, D), jnp.float32),
        pltpu.VMEM((1, GROUP), jnp.int32),
        pltpu.VMEM((1, GROUP), jnp.int32),
        pltpu.VMEM((1, GROUP), jnp.int32),
        pltpu.VMEM((1, GROUP), jnp.float32),
        pltpu.VMEM((GROUP, D), jnp.float32),
        pltpu.VMEM((CHUNK, D), jnp.float32),
        pltpu.VMEM((CHUNK, D), jnp.float32),
        pltpu.SemaphoreType.DMA,
    ]

    def body(x, prev, src3, dst3, w3, zeros, *rest):
        if last:
            out, acc, sidx, didx, draw, wv, rows, db, pb, sem = rest
        else:
            xn, sumn, acc, sidx, didx, draw, wv, rows, db, pb, sem = rest
        c = lax.axis_index("c")
        s = lax.axis_index("s")
        base = c * HALF

        qlo = s * CB + jnp.minimum(s, CR)
        qcnt = CB + jnp.where(s < CR, 1, 0)

        def zloop(q, _):
            off = q * CHUNK
            pltpu.sync_copy(zeros.at[pl.ds(off, CHUNK)],
                            acc.at[pl.ds(off, CHUNK)])
            return 0

        lax.fori_loop(qlo, qlo + qcnt, zloop, 0)
        plsc.subcore_barrier()

        glo = s * GB + jnp.minimum(s, GR)
        gcnt = GB + jnp.where(s < GR, 1, 0)

        def eloop(g, carry):
            pltpu.sync_copy(src3.at[g], sidx)
            pltpu.sync_copy(dst3.at[g], draw)
            pltpu.sync_copy(w3.at[g], wv)
            pltpu.async_copy(x.at[sidx.at[0]], rows, sem).wait()

            def cgrp(i, _):
                d16 = draw[0, pl.ds(i * 16, 16)] - base
                ok = (d16 >= 0) & (d16 < HALF)
                didx[0, pl.ds(i * 16, 16)] = jnp.where(ok, d16, HALF)
                w16 = wv[0, pl.ds(i * 16, 16)]
                for e in range(16):
                    idx_e = i * 16 + e
                    wsc = w16[e]
                    r0 = rows[idx_e, pl.ds(0, 16)]
                    rows[idx_e, pl.ds(0, 16)] = r0 * wsc
                    r1 = rows[idx_e, pl.ds(16, 16)]
                    rows[idx_e, pl.ds(16, 16)] = r1 * wsc
                return 0

            lax.fori_loop(0, GROUP // 16, cgrp, 0)
            pltpu.sync_copy(rows, acc.at[didx.at[0]], add=True)
            return carry

        lax.fori_loop(glo, glo + gcnt, eloop, 0)
        plsc.subcore_barrier()

        def dloop(q, _):
            off = q * CHUNK
            pltpu.sync_copy(acc.at[pl.ds(off, CHUNK)], db)
            if not last:
                pltpu.sync_copy(db, xn.at[pl.ds(base + off, CHUNK)])
            pltpu.sync_copy(prev.at[pl.ds(base + off, CHUNK)], pb)

            def rloop(r, _):
                for h in range(2):
                    v = db[r, pl.ds(16 * h, 16)] + pb[r, pl.ds(16 * h, 16)]
                    if last:
                        v = v * 0.25
                    db[r, pl.ds(16 * h, 16)] = v
                return 0

            lax.fori_loop(0, CHUNK, rloop, 0)
            if last:
                pltpu.sync_copy(db, out.at[pl.ds(base + off, CHUNK)])
            else:
                pltpu.sync_copy(db, sumn.at[pl.ds(base + off, CHUNK)])
            return 0

        lax.fori_loop(qlo, qlo + qcnt, dloop, 0)

    return pl.kernel(
        body, out_type=out_type, mesh=mesh, scratch_types=scratch,
        compiler_params=pltpu.CompilerParams(use_tc_tiling_on_sc=False))


_layer = _make_layer(last=False)
_layer_last = _make_layer(last=True)


def kernel(user_table, item_table, edge_index, edge_weight):
    all_emb = jnp.concatenate([user_table, item_table], axis=0)
    src3 = edge_index[0].reshape(NGROUPS, 1, GROUP)
    dst3 = edge_index[1].reshape(NGROUPS, 1, GROUP)
    w3 = edge_weight.reshape(NGROUPS, 1, GROUP)
    zeros = jnp.zeros((HALF, D), jnp.float32)
    x1, s1 = _layer(all_emb, all_emb, src3, dst3, w3, zeros)
    x2, s2 = _layer(x1, s1, src3, dst3, w3, zeros)
    out = _layer_last(x2, s2, src3, dst3, w3, zeros)
    return out[:N_USERS], out[N_USERS:]
